# baseline (speedup 1.0000x reference)
import functools

import jax
import jax.numpy as jnp
from jax import lax
from jax.experimental import pallas as pl
from jax.experimental.pallas import tpu as pltpu
from jax.experimental.pallas import tpu_sc as plsc

N = 10000
NPAD = 10240
E = 320000
ROW = 128
EROWS = 2560
NC, NS = 2, 16
NW = NC * NS
ROWS_PER_W = EROWS // NW
RPT = NPAD // NS


def _sc_mesh():
    return plsc.VectorSubcoreMesh(core_axis_name="c", subcore_axis_name="s")


_SC_PARAMS = pltpu.CompilerParams(use_tc_tiling_on_sc=False)


def _edge_pass(d):
    NBUF = 4
    NITER = ROWS_PER_W // NBUF
    ZR = RPT // 4

    @functools.partial(
        pl.kernel,
        out_type=jax.ShapeDtypeStruct((NC, NPAD, d), jnp.float32),
        mesh=_sc_mesh(),
        compiler_params=_SC_PARAMS,
        scratch_types=[
            pltpu.VMEM((ROWS_PER_W, ROW), jnp.int32),
            pltpu.VMEM((ROWS_PER_W, ROW), jnp.int32),
            pltpu.VMEM((NBUF, ROW, d), jnp.float32),
            pltpu.VMEM((ZR, d), jnp.float32),
            pltpu.VMEM_SHARED((NPAD, d), jnp.float32),
            [pltpu.SemaphoreType.DMA] * NBUF,
            [pltpu.SemaphoreType.DMA] * NBUF,
        ],
    )
    def kern(g_hbm, src_hbm, dst_hbm, out_hbm, src_v, dst_v, gbuf, zbuf, acc,
             gsems, ssems):
        c = lax.axis_index("c")
        s = lax.axis_index("s")
        wid = s * NC + c

        @pl.loop(0, ZR)
        def _(i):
            @pl.loop(0, d, step=16)
            def _(j):
                zbuf[i, pl.ds(j, 16)] = jnp.zeros((16,), jnp.float32)

        @pl.loop(0, RPT // ZR)
        def _(k):
            pltpu.sync_copy(zbuf, acc.at[pl.ds(s * RPT + k * ZR, ZR)])

        rb = wid * ROWS_PER_W
        pltpu.sync_copy(src_hbm.at[pl.ds(rb, ROWS_PER_W)], src_v)
        pltpu.sync_copy(dst_hbm.at[pl.ds(rb, ROWS_PER_W)], dst_v)
        plsc.subcore_barrier()

        for b in range(NBUF):
            pltpu.async_copy(g_hbm.at[src_v.at[b]], gbuf.at[b], gsems[b])

        @pl.loop(0, NITER)
        def _(it):
            j0 = it * NBUF
            for b in range(NBUF):
                j = j0 + b
                pltpu.make_async_copy(g_hbm.at[src_v.at[j]], gbuf.at[b],
                                      gsems[b]).wait()
                pltpu.async_copy(gbuf.at[b], acc.at[dst_v.at[j]], ssems[b],
                                 add=True)
            for b in range(NBUF):
                j = j0 + b
                pltpu.make_async_copy(gbuf.at[b], acc.at[dst_v.at[j]],
                                      ssems[b]).wait()

                @pl.when(j + NBUF < ROWS_PER_W)
                def _():
                    pltpu.async_copy(g_hbm.at[src_v.at[j + NBUF]], gbuf.at[b],
                                     gsems[b])

        plsc.subcore_barrier()
        pltpu.sync_copy(acc.at[pl.ds(s * RPT, RPT)],
                        out_hbm.at[c, pl.ds(s * RPT, RPT)])

    return kern


DEGW = 16


@functools.partial(
    pl.kernel,
    out_type=jax.ShapeDtypeStruct((NC, NPAD, DEGW), jnp.float32),
    mesh=_sc_mesh(),
    compiler_params=_SC_PARAMS,
    scratch_types=[
        pltpu.VMEM((ROWS_PER_W, ROW), jnp.int32),
        pltpu.VMEM((ROW, DEGW), jnp.float32),
        pltpu.VMEM((RPT, DEGW), jnp.float32),
        pltpu.VMEM_SHARED((NPAD, DEGW), jnp.float32),
    ],
)
def _deg_pass(dst_hbm, out_hbm, dst_v, ones_v, zbuf, acc):
    c = lax.axis_index("c")
    s = lax.axis_index("s")
    wid = s * NC + c

    @pl.loop(0, RPT)
    def _(i):
        zbuf[i, pl.ds(0, 16)] = jnp.zeros((16,), jnp.float32)

    @pl.loop(0, ROW)
    def _(i):
        ones_v[i, pl.ds(0, 16)] = jnp.ones((16,), jnp.float32)

    pltpu.sync_copy(zbuf, acc.at[pl.ds(s * RPT, RPT)])
    pltpu.sync_copy(dst_hbm.at[pl.ds(wid * ROWS_PER_W, ROWS_PER_W)], dst_v)
    plsc.subcore_barrier()

    @pl.loop(0, ROWS_PER_W)
    def _(j):
        pltpu.sync_copy(ones_v, acc.at[dst_v.at[j]], add=True)

    plsc.subcore_barrier()
    pltpu.sync_copy(acc.at[pl.ds(s * RPT, RPT)],
                    out_hbm.at[c, pl.ds(s * RPT, RPT)])


TCB = 1280
_G = NPAD // TCB


def _rows(block_cols):
    return pl.BlockSpec((TCB, block_cols), lambda i: (i, 0))


def _full(shape):
    return pl.BlockSpec(shape, lambda i: tuple(0 for _ in shape))


def _pair(d):
    return pl.BlockSpec((NC, TCB, d), lambda i: (0, i, 0))


def _tc(body, in_specs, out_shapes, out_specs):
    return pl.pallas_call(
        body,
        grid=(_G,),
        in_specs=in_specs,
        out_specs=out_specs,
        out_shape=out_shapes,
    )


def _mm_body(x_ref, w_ref, o_ref):
    o_ref[...] = jnp.dot(x_ref[...], w_ref[...],
                         preferred_element_type=jnp.float32)


def _stageA2_body(h_ref, cnt_ref, g_ref, dinv_ref):
    dinv = lax.rsqrt(1.0 + cnt_ref[0, :, 0:1] + cnt_ref[1, :, 0:1])
    dinv_ref[...] = dinv
    g_ref[...] = dinv * h_ref[...]


def _stageB_body(es_ref, g_ref, dinv_ref, b_ref, w_ref, o_ref):
    dinv = dinv_ref[...]
    z1 = jnp.maximum(
        dinv * (es_ref[0] + es_ref[1] + g_ref[...]) + b_ref[...], 0.0)
    o_ref[...] = dinv * jnp.dot(z1, w_ref[...],
                                preferred_element_type=jnp.float32)


def _stageC_body(es_ref, g_ref, dinv_ref, b_ref, z_ref, g3_ref):
    dinv = dinv_ref[...]
    z = dinv * (es_ref[0] + es_ref[1] + g_ref[...]) + b_ref[...]
    z_ref[...] = z
    g3_ref[...] = dinv * z


def _stageD_body(es_ref, g_ref, dinv_ref, w_ref, b_ref, o_ref):
    dinv = dinv_ref[...]
    pz = dinv * (es_ref[0] + es_ref[1] + g_ref[...])
    r3 = jnp.maximum(
        jnp.dot(pz, w_ref[...], preferred_element_type=jnp.float32)
        + b_ref[...], 0.0)
    o_ref[...] = dinv * r3


def _stageE_body(es_ref, g_ref, dinv_ref, w_ref, b_ref, o_ref):
    pr = dinv_ref[...] * (es_ref[0] + es_ref[1] + g_ref[...])
    o_ref[...] = jnp.dot(pr, w_ref[...],
                         preferred_element_type=jnp.float32) + b_ref[...]


def kernel(x, edge_index, W1, b1, W2, b2, W3, b3, W4, b4):
    f32 = jnp.float32
    src = edge_index[0].astype(jnp.int32)
    dst = edge_index[1].astype(jnp.int32)
    npad_e = EROWS * ROW - E
    src2d = jnp.concatenate(
        [src, jnp.full((npad_e,), N, jnp.int32)]).reshape(EROWS, ROW)
    dst2d = jnp.concatenate(
        [dst, jnp.full((npad_e,), N, jnp.int32)]).reshape(EROWS, ROW)
    x_p = jnp.zeros((NPAD, 128), f32).at[:N].set(x)
    b1r, b2r = b1.reshape(1, 64), b2.reshape(1, 32)
    b3r, b4r = b3.reshape(1, 64), b4.reshape(1, 128)

    cnt = _deg_pass(dst2d)
    h1 = _tc(_mm_body, [_rows(128), _full((128, 64))],
             jax.ShapeDtypeStruct((NPAD, 64), f32), _rows(64))(x_p, W1)

    g1, dinv = _tc(
        _stageA2_body, [_rows(64), _pair(DEGW)],
        (jax.ShapeDtypeStruct((NPAD, 64), f32),
         jax.ShapeDtypeStruct((NPAD, 1), f32)),
        (_rows(64), _rows(1)))(h1, cnt)

    es1 = _edge_pass(64)(g1, src2d, dst2d)
    g2 = _tc(
        _stageB_body,
        [_pair(64), _rows(64), _rows(1), _full((1, 64)), _full((64, 32))],
        jax.ShapeDtypeStruct((NPAD, 32), f32),
        _rows(32))(es1, g1, dinv, b1r, W2)

    es2 = _edge_pass(32)(g2, src2d, dst2d)
    z_p, g3 = _tc(
        _stageC_body,
        [_pair(32), _rows(32), _rows(1), _full((1, 32))],
        (jax.ShapeDtypeStruct((NPAD, 32), f32),
         jax.ShapeDtypeStruct((NPAD, 32), f32)),
        (_rows(32), _rows(32)))(es2, g2, dinv, b2r)

    es3 = _edge_pass(32)(g3, src2d, dst2d)
    g4 = _tc(
        _stageD_body,
        [_pair(32), _rows(32), _rows(1), _full((32, 64)), _full((1, 64))],
        jax.ShapeDtypeStruct((NPAD, 64), f32),
        _rows(64))(es3, g3, dinv, W3, b3r)

    es4 = _edge_pass(64)(g4, src2d, dst2d)
    x_hat_p = _tc(
        _stageE_body,
        [_pair(64), _rows(64), _rows(1), _full((64, 128)), _full((1, 128))],
        jax.ShapeDtypeStruct((NPAD, 128), f32),
        _rows(128))(es4, g4, dinv, W4, b4r)

    return (x_hat_p[:N], z_p[:N])


# speedup vs baseline: 2.0311x; 2.0311x over previous
import functools

import jax
import jax.numpy as jnp
from jax import lax
from jax.experimental import pallas as pl
from jax.experimental.pallas import tpu as pltpu
from jax.experimental.pallas import tpu_sc as plsc

N = 10000
NPAD = 10240
E = 320000
ROW = 128
EROWS = 2560
NC, NS = 2, 16
NW = NC * NS
ROWS_PER_W = EROWS // NW
RPT = NPAD // NS


def _sc_mesh():
    return plsc.VectorSubcoreMesh(core_axis_name="c", subcore_axis_name="s")


_SC_PARAMS = pltpu.CompilerParams(use_tc_tiling_on_sc=False)


def _edge_pass(d):
    NBUF = 2
    NITER = ROWS_PER_W // NBUF
    ZR = RPT // 4

    @functools.partial(
        pl.kernel,
        out_type=jax.ShapeDtypeStruct((NC, NPAD, d), jnp.float32),
        mesh=_sc_mesh(),
        compiler_params=_SC_PARAMS,
        scratch_types=[
            pltpu.VMEM((ROWS_PER_W, ROW), jnp.int32),
            pltpu.VMEM((ROWS_PER_W, ROW), jnp.int32),
            pltpu.VMEM((NBUF, ROW, d), jnp.float32),
            pltpu.VMEM((ZR, d), jnp.float32),
            pltpu.VMEM_SHARED((NPAD, d), jnp.float32),
            pltpu.VMEM_SHARED((NPAD, d), jnp.float32),
            [pltpu.SemaphoreType.DMA] * NBUF,
            [pltpu.SemaphoreType.DMA] * NBUF,
        ],
    )
    def kern(g_hbm, src_hbm, dst_hbm, out_hbm, src_v, dst_v, gbuf, zbuf, gtab,
             acc, gsems, ssems):
        c = lax.axis_index("c")
        s = lax.axis_index("s")
        wid = s * NC + c

        pltpu.sync_copy(g_hbm.at[pl.ds(s * RPT, RPT)],
                        gtab.at[pl.ds(s * RPT, RPT)])

        @pl.loop(0, ZR)
        def _(i):
            @pl.loop(0, d, step=16)
            def _(j):
                zbuf[i, pl.ds(j, 16)] = jnp.zeros((16,), jnp.float32)

        @pl.loop(0, RPT // ZR)
        def _(k):
            pltpu.sync_copy(zbuf, acc.at[pl.ds(s * RPT + k * ZR, ZR)])

        rb = wid * ROWS_PER_W
        pltpu.sync_copy(src_hbm.at[pl.ds(rb, ROWS_PER_W)], src_v)
        pltpu.sync_copy(dst_hbm.at[pl.ds(rb, ROWS_PER_W)], dst_v)
        plsc.subcore_barrier()

        for b in range(NBUF):
            pltpu.async_copy(gtab.at[src_v.at[b]], gbuf.at[b], gsems[b])

        @pl.loop(0, NITER)
        def _(it):
            j0 = it * NBUF
            for b in range(NBUF):
                j = j0 + b
                pltpu.make_async_copy(gtab.at[src_v.at[j]], gbuf.at[b],
                                      gsems[b]).wait()
                pltpu.async_copy(gbuf.at[b], acc.at[dst_v.at[j]], ssems[b],
                                 add=True)
            for b in range(NBUF):
                j = j0 + b
                pltpu.make_async_copy(gbuf.at[b], acc.at[dst_v.at[j]],
                                      ssems[b]).wait()

                @pl.when(j + NBUF < ROWS_PER_W)
                def _():
                    pltpu.async_copy(gtab.at[src_v.at[j + NBUF]], gbuf.at[b],
                                     gsems[b])

        plsc.subcore_barrier()
        pltpu.sync_copy(acc.at[pl.ds(s * RPT, RPT)],
                        out_hbm.at[c, pl.ds(s * RPT, RPT)])

    return kern


DEGW = 16


@functools.partial(
    pl.kernel,
    out_type=jax.ShapeDtypeStruct((NC, NPAD, DEGW), jnp.float32),
    mesh=_sc_mesh(),
    compiler_params=_SC_PARAMS,
    scratch_types=[
        pltpu.VMEM((ROWS_PER_W, ROW), jnp.int32),
        pltpu.VMEM((ROW, DEGW), jnp.float32),
        pltpu.VMEM((RPT, DEGW), jnp.float32),
        pltpu.VMEM_SHARED((NPAD, DEGW), jnp.float32),
    ],
)
def _deg_pass(dst_hbm, out_hbm, dst_v, ones_v, zbuf, acc):
    c = lax.axis_index("c")
    s = lax.axis_index("s")
    wid = s * NC + c

    @pl.loop(0, RPT)
    def _(i):
        zbuf[i, pl.ds(0, 16)] = jnp.zeros((16,), jnp.float32)

    @pl.loop(0, ROW)
    def _(i):
        ones_v[i, pl.ds(0, 16)] = jnp.ones((16,), jnp.float32)

    pltpu.sync_copy(zbuf, acc.at[pl.ds(s * RPT, RPT)])
    pltpu.sync_copy(dst_hbm.at[pl.ds(wid * ROWS_PER_W, ROWS_PER_W)], dst_v)
    plsc.subcore_barrier()

    @pl.loop(0, ROWS_PER_W)
    def _(j):
        pltpu.sync_copy(ones_v, acc.at[dst_v.at[j]], add=True)

    plsc.subcore_barrier()
    pltpu.sync_copy(acc.at[pl.ds(s * RPT, RPT)],
                    out_hbm.at[c, pl.ds(s * RPT, RPT)])


TCB = 1280
_G = NPAD // TCB


def _rows(block_cols):
    return pl.BlockSpec((TCB, block_cols), lambda i: (i, 0))


def _full(shape):
    return pl.BlockSpec(shape, lambda i: tuple(0 for _ in shape))


def _pair(d):
    return pl.BlockSpec((NC, TCB, d), lambda i: (0, i, 0))


def _tc(body, in_specs, out_shapes, out_specs):
    return pl.pallas_call(
        body,
        grid=(_G,),
        in_specs=in_specs,
        out_specs=out_specs,
        out_shape=out_shapes,
    )


def _mm_body(x_ref, w_ref, o_ref):
    o_ref[...] = jnp.dot(x_ref[...], w_ref[...],
                         preferred_element_type=jnp.float32)


def _stageA2_body(h_ref, cnt_ref, g_ref, dinv_ref):
    dinv = lax.rsqrt(1.0 + cnt_ref[0, :, 0:1] + cnt_ref[1, :, 0:1])
    dinv_ref[...] = dinv
    g_ref[...] = dinv * h_ref[...]


def _stageB_body(es_ref, g_ref, dinv_ref, b_ref, w_ref, o_ref):
    dinv = dinv_ref[...]
    z1 = jnp.maximum(
        dinv * (es_ref[0] + es_ref[1] + g_ref[...]) + b_ref[...], 0.0)
    o_ref[...] = dinv * jnp.dot(z1, w_ref[...],
                                preferred_element_type=jnp.float32)


def _stageC_body(es_ref, g_ref, dinv_ref, b_ref, z_ref, g3_ref):
    dinv = dinv_ref[...]
    z = dinv * (es_ref[0] + es_ref[1] + g_ref[...]) + b_ref[...]
    z_ref[...] = z
    g3_ref[...] = dinv * z


def _stageD_body(es_ref, g_ref, dinv_ref, w_ref, b_ref, o_ref):
    dinv = dinv_ref[...]
    pz = dinv * (es_ref[0] + es_ref[1] + g_ref[...])
    r3 = jnp.maximum(
        jnp.dot(pz, w_ref[...], preferred_element_type=jnp.float32)
        + b_ref[...], 0.0)
    o_ref[...] = dinv * r3


def _stageE_body(es_ref, g_ref, dinv_ref, w_ref, b_ref, o_ref):
    pr = dinv_ref[...] * (es_ref[0] + es_ref[1] + g_ref[...])
    o_ref[...] = jnp.dot(pr, w_ref[...],
                         preferred_element_type=jnp.float32) + b_ref[...]


def kernel(x, edge_index, W1, b1, W2, b2, W3, b3, W4, b4):
    f32 = jnp.float32
    src = edge_index[0].astype(jnp.int32)
    dst = edge_index[1].astype(jnp.int32)
    npad_e = EROWS * ROW - E
    src2d = jnp.concatenate(
        [src, jnp.full((npad_e,), N, jnp.int32)]).reshape(EROWS, ROW)
    dst2d = jnp.concatenate(
        [dst, jnp.full((npad_e,), N, jnp.int32)]).reshape(EROWS, ROW)
    x_p = jnp.zeros((NPAD, 128), f32).at[:N].set(x)
    b1r, b2r = b1.reshape(1, 64), b2.reshape(1, 32)
    b3r, b4r = b3.reshape(1, 64), b4.reshape(1, 128)

    cnt = _deg_pass(dst2d)
    h1 = _tc(_mm_body, [_rows(128), _full((128, 64))],
             jax.ShapeDtypeStruct((NPAD, 64), f32), _rows(64))(x_p, W1)

    g1, dinv = _tc(
        _stageA2_body, [_rows(64), _pair(DEGW)],
        (jax.ShapeDtypeStruct((NPAD, 64), f32),
         jax.ShapeDtypeStruct((NPAD, 1), f32)),
        (_rows(64), _rows(1)))(h1, cnt)

    es1 = _edge_pass(64)(g1, src2d, dst2d)
    g2 = _tc(
        _stageB_body,
        [_pair(64), _rows(64), _rows(1), _full((1, 64)), _full((64, 32))],
        jax.ShapeDtypeStruct((NPAD, 32), f32),
        _rows(32))(es1, g1, dinv, b1r, W2)

    es2 = _edge_pass(32)(g2, src2d, dst2d)
    z_p, g3 = _tc(
        _stageC_body,
        [_pair(32), _rows(32), _rows(1), _full((1, 32))],
        (jax.ShapeDtypeStruct((NPAD, 32), f32),
         jax.ShapeDtypeStruct((NPAD, 32), f32)),
        (_rows(32), _rows(32)))(es2, g2, dinv, b2r)

    es3 = _edge_pass(32)(g3, src2d, dst2d)
    g4 = _tc(
        _stageD_body,
        [_pair(32), _rows(32), _rows(1), _full((32, 64)), _full((1, 64))],
        jax.ShapeDtypeStruct((NPAD, 64), f32),
        _rows(64))(es3, g3, dinv, W3, b3r)

    es4 = _edge_pass(64)(g4, src2d, dst2d)
    x_hat_p = _tc(
        _stageE_body,
        [_pair(64), _rows(64), _rows(1), _full((64, 128)), _full((1, 128))],
        jax.ShapeDtypeStruct((NPAD, 128), f32),
        _rows(128))(es4, g4, dinv, W4, b4r)

    return (x_hat_p[:N], z_p[:N])
